# baseline (speedup 1.0000x reference)
import functools

import jax
import jax.numpy as jnp
from jax import lax
from jax.experimental import pallas as pl
from jax.experimental.pallas import tpu as pltpu
from jax.experimental.pallas import tpu_sc as plsc

N = 10000
E = 320000
D = 128
NP = 10240
NW = 32
CHUNK = 64
EPW = 10240
EPAD = NW * EPW - E
NCHUNKS = EPW // CHUNK
NPAIRS = NCHUNKS // 2
ROWS_PER_TILE = NP // 16
L = 16


def _edge_body(h_hbm, idx_hbm, attr_hbm, zeros_hbm, out_hbm,
               acc, idx_all, rows_v, attr_v, src_u, dst_u,
               sem_ga, sem_gb, sem_aa, sem_ab, sem_sa, sem_sb):
    cid = lax.axis_index("c")
    sid = lax.axis_index("s")
    wid = cid * 16 + sid

    pltpu.sync_copy(zeros_hbm.at[pl.ds(sid * ROWS_PER_TILE, ROWS_PER_TILE)],
                    acc.at[pl.ds(sid * ROWS_PER_TILE, ROWS_PER_TILE)])

    pltpu.sync_copy(idx_hbm.at[wid], idx_all)
    plsc.subcore_barrier()

    ebase = wid * EPW

    def unpack_idx(i, slot):
        row = i // 2
        col = (i % 2) * CHUNK
        for v in range(CHUNK // L):
            p = idx_all[row, pl.ds(col + v * L, L)]
            src_u[slot, pl.ds(v * L, L)] = lax.shift_right_logical(p, 16)
            dst_u[slot, pl.ds(v * L, L)] = jnp.bitwise_and(p, 0xFFFF)

    def compute(slot):
        def row_body(r, c2):
            for rr in range(4):
                for cc in range(D // 16):
                    sl = pl.ds(cc * 16, 16)
                    v = rows_v[slot, 4 * r + rr, sl] + attr_v[slot, 4 * r + rr, sl]
                    rows_v[slot, 4 * r + rr, sl] = jnp.maximum(v, 0.0)
            return c2
        lax.fori_loop(0, CHUNK // 4, row_body, 0)

    def prefetch(i, slot, sem_g, sem_a):
        unpack_idx(i, slot)
        abase = jnp.minimum(ebase + i * CHUNK, E - CHUNK)
        pltpu.async_copy(attr_hbm.at[pl.ds(abase, CHUNK)], attr_v.at[slot], sem_a)
        pltpu.async_copy(h_hbm.at[src_u.at[slot]], rows_v.at[slot], sem_g)

    def wait_data(slot, sem_g, sem_a):
        pltpu.make_async_copy(attr_hbm.at[pl.ds(0, CHUNK)],
                              attr_v.at[slot], sem_a).wait()
        pltpu.make_async_copy(attr_hbm.at[pl.ds(0, CHUNK)],
                              rows_v.at[slot], sem_g).wait()

    def drain_scatter(slot, sem):
        pltpu.make_async_copy(attr_hbm.at[pl.ds(0, CHUNK)],
                              rows_v.at[slot], sem).wait()

    prefetch(0, 0, sem_ga, sem_aa)

    def pair_body(j, carry):
        b = 2 * j + 1
        @pl.when(j > 0)
        def _():
            drain_scatter(1, sem_sb)
        prefetch(b, 1, sem_gb, sem_ab)
        wait_data(0, sem_ga, sem_aa)
        compute(0)
        pltpu.async_copy(rows_v.at[0], acc.at[dst_u.at[0]], sem_sa, add=True)
        wait_data(1, sem_gb, sem_ab)
        compute(1)
        pltpu.async_copy(rows_v.at[1], acc.at[dst_u.at[1]], sem_sb, add=True)
        @pl.when(j < NPAIRS - 1)
        def _():
            drain_scatter(0, sem_sa)
            prefetch(2 * j + 2, 0, sem_ga, sem_aa)
        return carry

    lax.fori_loop(0, NPAIRS, pair_body, 0)
    drain_scatter(0, sem_sa)
    drain_scatter(1, sem_sb)
    plsc.subcore_barrier()

    pltpu.sync_copy(acc.at[pl.ds(sid * ROWS_PER_TILE, ROWS_PER_TILE)],
                    out_hbm.at[cid, pl.ds(sid * ROWS_PER_TILE, ROWS_PER_TILE)])


@jax.jit
def _edge_aggregate(h, idx, edge_attr, zeros):
    mesh = plsc.VectorSubcoreMesh(core_axis_name="c", subcore_axis_name="s")
    return pl.kernel(
        _edge_body,
        out_type=jax.ShapeDtypeStruct((2, NP, D), jnp.float32),
        mesh=mesh,
        scratch_types=[
            pltpu.VMEM_SHARED((NP, D), jnp.float32),
            pltpu.VMEM((EPW // 128, 128), jnp.int32),
            pltpu.VMEM((2, CHUNK, D), jnp.float32),
            pltpu.VMEM((2, CHUNK, D), jnp.float32),
            pltpu.VMEM((2, CHUNK), jnp.int32),
            pltpu.VMEM((2, CHUNK), jnp.int32),
            pltpu.SemaphoreType.DMA,
            pltpu.SemaphoreType.DMA,
            pltpu.SemaphoreType.DMA,
            pltpu.SemaphoreType.DMA,
            pltpu.SemaphoreType.DMA,
            pltpu.SemaphoreType.DMA,
        ],
    )(h, idx, edge_attr, zeros)


def _mlp_body(h_ref, a0_ref, a1_ref, w1_ref, b1_ref, g1_ref, be1_ref,
              w2_ref, b2_ref, g2_ref, be2_ref, out_ref):
    z = h_ref[...] + a0_ref[...] + a1_ref[...]
    z = jnp.dot(z, w1_ref[...], preferred_element_type=jnp.float32) + b1_ref[...]
    mu = jnp.mean(z, axis=0, keepdims=True)
    var = jnp.mean((z - mu) * (z - mu), axis=0, keepdims=True)
    z = g1_ref[...] * (z - mu) / jnp.sqrt(var + 1e-5) + be1_ref[...]
    z = jnp.maximum(z, 0.0)
    z = jnp.dot(z, w2_ref[...], preferred_element_type=jnp.float32) + b2_ref[...]
    z = jnp.maximum(z, 0.0)
    mu = jnp.mean(z, axis=0, keepdims=True)
    var = jnp.mean((z - mu) * (z - mu), axis=0, keepdims=True)
    z = g2_ref[...] * (z - mu) / jnp.sqrt(var + 1e-5) + be2_ref[...]
    out_ref[...] = jnp.maximum(z, 0.0)


@jax.jit
def _mlp(h, a0, a1, w1, b1, g1, be1, w2, b2, g2, be2):
    return pl.pallas_call(
        _mlp_body,
        out_shape=jax.ShapeDtypeStruct((N, D), jnp.float32),
    )(h, a0, a1, w1, b1, g1, be1, w2, b2, g2, be2)


def kernel(x, edge_index, edge_attr, params):
    ei = edge_index.astype(jnp.int32)
    packed = jnp.concatenate(
        [(ei[0] << 16) | ei[1],
         jnp.full((EPAD,), NP - 1, jnp.int32)])
    packed = packed.reshape(NW, EPW // 128, 128)
    zeros = jnp.zeros((NP, D), jnp.float32)
    h = x
    for p in params:
        parts = _edge_aggregate(h, packed, edge_attr, zeros)
        h = _mlp(h, parts[0, :N], parts[1, :N],
                 p['W1'], p['b1'].reshape(1, D), p['g1'].reshape(1, D),
                 p['be1'].reshape(1, D),
                 p['W2'], p['b2'].reshape(1, D), p['g2'].reshape(1, D),
                 p['be2'].reshape(1, D))
    return h


# speedup vs baseline: 1.2475x; 1.2475x over previous
import functools

import jax
import jax.numpy as jnp
from jax import lax
from jax.experimental import pallas as pl
from jax.experimental.pallas import tpu as pltpu
from jax.experimental.pallas import tpu_sc as plsc

N = 10000
E = 320000
D = 128
NP = 10240
NW = 32
CHUNK = 125
EPW = E // NW
NCHUNKS = EPW // CHUNK
ROWS_PER_TILE = NP // 16


def _edge_body(h_hbm, src_hbm, dst_hbm, attr_hbm, zeros_hbm, out_hbm,
               acc, src_v, dst_v, rows_v, attr_v, sem):
    cid = lax.axis_index("c")
    sid = lax.axis_index("s")
    wid = cid * 16 + sid

    pltpu.sync_copy(zeros_hbm.at[pl.ds(sid * ROWS_PER_TILE, ROWS_PER_TILE)],
                    acc.at[pl.ds(sid * ROWS_PER_TILE, ROWS_PER_TILE)])
    plsc.subcore_barrier()

    def chunk_body(i, carry):
        pltpu.sync_copy(src_hbm.at[wid, i], src_v)
        pltpu.sync_copy(dst_hbm.at[wid, i], dst_v)
        pltpu.async_copy(h_hbm.at[src_v], rows_v, sem).wait()
        pltpu.sync_copy(attr_hbm.at[wid, i], attr_v)

        def row_body(r, c2):
            for cc in range(D // 16):
                sl = pl.ds(cc * 16, 16)
                v = rows_v[r, sl] + attr_v[r, sl]
                rows_v[r, sl] = jnp.maximum(v, 0.0)
            return c2

        lax.fori_loop(0, CHUNK, row_body, 0)
        pltpu.sync_copy(rows_v, acc.at[dst_v], add=True)
        return carry

    lax.fori_loop(0, NCHUNKS, chunk_body, 0)
    plsc.subcore_barrier()

    pltpu.sync_copy(acc.at[pl.ds(sid * ROWS_PER_TILE, ROWS_PER_TILE)],
                    out_hbm.at[cid, pl.ds(sid * ROWS_PER_TILE, ROWS_PER_TILE)])


@jax.jit
def _edge_aggregate(h, src, dst, edge_attr, zeros):
    mesh = plsc.VectorSubcoreMesh(core_axis_name="c", subcore_axis_name="s")
    return pl.kernel(
        _edge_body,
        out_type=jax.ShapeDtypeStruct((2, NP, D), jnp.float32),
        mesh=mesh,
        scratch_types=[
            pltpu.VMEM_SHARED((NP, D), jnp.float32),
            pltpu.VMEM((CHUNK,), jnp.int32),
            pltpu.VMEM((CHUNK,), jnp.int32),
            pltpu.VMEM((CHUNK, D), jnp.float32),
            pltpu.VMEM((CHUNK, D), jnp.float32),
            pltpu.SemaphoreType.DMA,
        ],
    )(h, src, dst, edge_attr, zeros)


def _mlp_body(h_ref, a0_ref, a1_ref, w1_ref, b1_ref, g1_ref, be1_ref,
              w2_ref, b2_ref, g2_ref, be2_ref, out_ref):
    z = h_ref[...] + a0_ref[...] + a1_ref[...]
    z = jnp.dot(z, w1_ref[...], preferred_element_type=jnp.float32) + b1_ref[...]
    mu = jnp.mean(z, axis=0, keepdims=True)
    var = jnp.mean((z - mu) * (z - mu), axis=0, keepdims=True)
    z = g1_ref[...] * (z - mu) / jnp.sqrt(var + 1e-5) + be1_ref[...]
    z = jnp.maximum(z, 0.0)
    z = jnp.dot(z, w2_ref[...], preferred_element_type=jnp.float32) + b2_ref[...]
    z = jnp.maximum(z, 0.0)
    mu = jnp.mean(z, axis=0, keepdims=True)
    var = jnp.mean((z - mu) * (z - mu), axis=0, keepdims=True)
    z = g2_ref[...] * (z - mu) / jnp.sqrt(var + 1e-5) + be2_ref[...]
    out_ref[...] = jnp.maximum(z, 0.0)


@jax.jit
def _mlp(h, a0, a1, w1, b1, g1, be1, w2, b2, g2, be2):
    return pl.pallas_call(
        _mlp_body,
        out_shape=jax.ShapeDtypeStruct((N, D), jnp.float32),
    )(h, a0, a1, w1, b1, g1, be1, w2, b2, g2, be2)


def kernel(x, edge_index, edge_attr, params):
    src = edge_index[0].astype(jnp.int32).reshape(NW, NCHUNKS, CHUNK)
    dst = edge_index[1].astype(jnp.int32).reshape(NW, NCHUNKS, CHUNK)
    attr = edge_attr.reshape(NW, NCHUNKS, CHUNK, D)
    zeros = jnp.zeros((NP, D), jnp.float32)
    h = x
    for p in params:
        parts = _edge_aggregate(h, src, dst, attr, zeros)
        h = _mlp(h, parts[0, :N], parts[1, :N],
                 p['W1'], p['b1'].reshape(1, D), p['g1'].reshape(1, D),
                 p['be1'].reshape(1, D),
                 p['W2'], p['b2'].reshape(1, D), p['g2'].reshape(1, D),
                 p['be2'].reshape(1, D))
    return h


# speedup vs baseline: 1.4005x; 1.1227x over previous
import functools

import jax
import jax.numpy as jnp
from jax import lax
from jax.experimental import pallas as pl
from jax.experimental.pallas import tpu as pltpu
from jax.experimental.pallas import tpu_sc as plsc

N = 10000
E = 320000
D = 128
NP = 10240
NW = 32
CHUNK = 125
EPW = E // NW
NCHUNKS = EPW // CHUNK
ROWS_PER_TILE = NP // 16


def _edge_body(h_hbm, src_hbm, dst_hbm, attr_hbm, zeros_hbm, out_hbm,
               acc, src_v, dst_v, rows_v, attr_v, sem):
    cid = lax.axis_index("c")
    sid = lax.axis_index("s")
    wid = cid * 16 + sid

    pltpu.sync_copy(zeros_hbm.at[pl.ds(sid * ROWS_PER_TILE, ROWS_PER_TILE)],
                    acc.at[pl.ds(sid * ROWS_PER_TILE, ROWS_PER_TILE)])
    plsc.subcore_barrier()

    def chunk_body(i, carry):
        pltpu.sync_copy(src_hbm.at[wid, i], src_v)
        pltpu.sync_copy(dst_hbm.at[wid, i], dst_v)
        g = pltpu.async_copy(h_hbm.at[src_v], rows_v, sem)
        pltpu.sync_copy(attr_hbm.at[wid, i], attr_v)
        g.wait()

        def row_body(r, c2):
            for cc in range(D // 16):
                sl = pl.ds(cc * 16, 16)
                v = rows_v[r, sl] + attr_v[r, sl]
                rows_v[r, sl] = jnp.maximum(v, 0.0)
            return c2

        lax.fori_loop(0, CHUNK, row_body, 0)
        pltpu.sync_copy(rows_v, acc.at[dst_v], add=True)
        return carry

    lax.fori_loop(0, NCHUNKS, chunk_body, 0)
    plsc.subcore_barrier()

    pltpu.sync_copy(acc.at[pl.ds(sid * ROWS_PER_TILE, ROWS_PER_TILE)],
                    out_hbm.at[cid, pl.ds(sid * ROWS_PER_TILE, ROWS_PER_TILE)])


@jax.jit
def _edge_aggregate(h, src, dst, edge_attr, zeros):
    mesh = plsc.VectorSubcoreMesh(core_axis_name="c", subcore_axis_name="s")
    return pl.kernel(
        _edge_body,
        out_type=jax.ShapeDtypeStruct((2, NP, D), jnp.float32),
        mesh=mesh,
        scratch_types=[
            pltpu.VMEM_SHARED((NP, D), jnp.float32),
            pltpu.VMEM((CHUNK,), jnp.int32),
            pltpu.VMEM((CHUNK,), jnp.int32),
            pltpu.VMEM((CHUNK, D), jnp.float32),
            pltpu.VMEM((CHUNK, D), jnp.float32),
            pltpu.SemaphoreType.DMA,
        ],
    )(h, src, dst, edge_attr, zeros)


def _mlp_body(h_ref, a0_ref, a1_ref, w1_ref, b1_ref, g1_ref, be1_ref,
              w2_ref, b2_ref, g2_ref, be2_ref, out_ref):
    z = h_ref[...] + a0_ref[...] + a1_ref[...]
    z = jnp.dot(z, w1_ref[...], preferred_element_type=jnp.float32) + b1_ref[...]
    mu = jnp.mean(z, axis=0, keepdims=True)
    var = jnp.mean((z - mu) * (z - mu), axis=0, keepdims=True)
    z = g1_ref[...] * (z - mu) / jnp.sqrt(var + 1e-5) + be1_ref[...]
    z = jnp.maximum(z, 0.0)
    z = jnp.dot(z, w2_ref[...], preferred_element_type=jnp.float32) + b2_ref[...]
    z = jnp.maximum(z, 0.0)
    mu = jnp.mean(z, axis=0, keepdims=True)
    var = jnp.mean((z - mu) * (z - mu), axis=0, keepdims=True)
    z = g2_ref[...] * (z - mu) / jnp.sqrt(var + 1e-5) + be2_ref[...]
    out_ref[...] = jnp.maximum(z, 0.0)


@jax.jit
def _mlp(h, a0, a1, w1, b1, g1, be1, w2, b2, g2, be2):
    return pl.pallas_call(
        _mlp_body,
        out_shape=jax.ShapeDtypeStruct((N, D), jnp.float32),
    )(h, a0, a1, w1, b1, g1, be1, w2, b2, g2, be2)


def kernel(x, edge_index, edge_attr, params):
    src = edge_index[0].astype(jnp.int32).reshape(NW, NCHUNKS, CHUNK)
    dst = edge_index[1].astype(jnp.int32).reshape(NW, NCHUNKS, CHUNK)
    attr = edge_attr.reshape(NW, NCHUNKS, CHUNK, D)
    zeros = jnp.zeros((NP, D), jnp.float32)
    h = x
    for p in params:
        parts = _edge_aggregate(h, src, dst, attr, zeros)
        h = _mlp(h, parts[0, :N], parts[1, :N],
                 p['W1'], p['b1'].reshape(1, D), p['g1'].reshape(1, D),
                 p['be1'].reshape(1, D),
                 p['W2'], p['b2'].reshape(1, D), p['g2'].reshape(1, D),
                 p['be2'].reshape(1, D))
    return h


# speedup vs baseline: 1.5562x; 1.1112x over previous
import functools

import jax
import jax.numpy as jnp
from jax import lax
from jax.experimental import pallas as pl
from jax.experimental.pallas import tpu as pltpu
from jax.experimental.pallas import tpu_sc as plsc

N = 10000
E = 320000
D = 128
NP = 10240
NW = 32
CHUNK = 125
EPW = E // NW
NCHUNKS = EPW // CHUNK
ROWS_PER_TILE = NP // 16


def _edge_body(h_hbm, src_hbm, dst_hbm, attr_hbm, zeros_hbm, out_hbm,
               acc, src_v, dst_v, rows_v, attr_v, sem, sem_s):
    cid = lax.axis_index("c")
    sid = lax.axis_index("s")
    wid = cid * 16 + sid

    pltpu.sync_copy(zeros_hbm.at[pl.ds(sid * ROWS_PER_TILE, ROWS_PER_TILE)],
                    acc.at[pl.ds(sid * ROWS_PER_TILE, ROWS_PER_TILE)])
    plsc.subcore_barrier()

    def chunk_body(i, carry):
        pltpu.sync_copy(src_hbm.at[wid, i], src_v)
        pltpu.sync_copy(dst_hbm.at[wid, i], dst_v)

        @pl.when(i > 0)
        def _():
            pltpu.make_async_copy(attr_hbm.at[wid, 0], rows_v, sem_s).wait()

        g = pltpu.async_copy(h_hbm.at[src_v], rows_v, sem)
        pltpu.sync_copy(attr_hbm.at[wid, i], attr_v)
        g.wait()

        def row_body(r, c2):
            for cc in range(D // 16):
                sl = pl.ds(cc * 16, 16)
                v = rows_v[r, sl] + attr_v[r, sl]
                rows_v[r, sl] = jnp.maximum(v, 0.0)
            return c2

        lax.fori_loop(0, CHUNK, row_body, 0)
        pltpu.async_copy(rows_v, acc.at[dst_v], sem_s, add=True)
        return carry

    lax.fori_loop(0, NCHUNKS, chunk_body, 0)
    pltpu.make_async_copy(attr_hbm.at[wid, 0], rows_v, sem_s).wait()
    plsc.subcore_barrier()

    pltpu.sync_copy(acc.at[pl.ds(sid * ROWS_PER_TILE, ROWS_PER_TILE)],
                    out_hbm.at[cid, pl.ds(sid * ROWS_PER_TILE, ROWS_PER_TILE)])


@jax.jit
def _edge_aggregate(h, src, dst, edge_attr, zeros):
    mesh = plsc.VectorSubcoreMesh(core_axis_name="c", subcore_axis_name="s")
    return pl.kernel(
        _edge_body,
        out_type=jax.ShapeDtypeStruct((2, NP, D), jnp.float32),
        mesh=mesh,
        scratch_types=[
            pltpu.VMEM_SHARED((NP, D), jnp.float32),
            pltpu.VMEM((CHUNK,), jnp.int32),
            pltpu.VMEM((CHUNK,), jnp.int32),
            pltpu.VMEM((CHUNK, D), jnp.float32),
            pltpu.VMEM((CHUNK, D), jnp.float32),
            pltpu.SemaphoreType.DMA,
            pltpu.SemaphoreType.DMA,
        ],
    )(h, src, dst, edge_attr, zeros)


def _mlp_body(h_ref, a0_ref, a1_ref, w1_ref, b1_ref, g1_ref, be1_ref,
              w2_ref, b2_ref, g2_ref, be2_ref, out_ref):
    z = h_ref[...] + a0_ref[...] + a1_ref[...]
    z = jnp.dot(z, w1_ref[...], preferred_element_type=jnp.float32) + b1_ref[...]
    mu = jnp.mean(z, axis=0, keepdims=True)
    var = jnp.mean((z - mu) * (z - mu), axis=0, keepdims=True)
    z = g1_ref[...] * (z - mu) / jnp.sqrt(var + 1e-5) + be1_ref[...]
    z = jnp.maximum(z, 0.0)
    z = jnp.dot(z, w2_ref[...], preferred_element_type=jnp.float32) + b2_ref[...]
    z = jnp.maximum(z, 0.0)
    mu = jnp.mean(z, axis=0, keepdims=True)
    var = jnp.mean((z - mu) * (z - mu), axis=0, keepdims=True)
    z = g2_ref[...] * (z - mu) / jnp.sqrt(var + 1e-5) + be2_ref[...]
    out_ref[...] = jnp.maximum(z, 0.0)


@jax.jit
def _mlp(h, a0, a1, w1, b1, g1, be1, w2, b2, g2, be2):
    return pl.pallas_call(
        _mlp_body,
        out_shape=jax.ShapeDtypeStruct((N, D), jnp.float32),
    )(h, a0, a1, w1, b1, g1, be1, w2, b2, g2, be2)


def kernel(x, edge_index, edge_attr, params):
    src = edge_index[0].astype(jnp.int32).reshape(NW, NCHUNKS, CHUNK)
    dst = edge_index[1].astype(jnp.int32).reshape(NW, NCHUNKS, CHUNK)
    attr = edge_attr.reshape(NW, NCHUNKS, CHUNK, D)
    zeros = jnp.zeros((NP, D), jnp.float32)
    h = x
    for p in params:
        parts = _edge_aggregate(h, src, dst, attr, zeros)
        h = _mlp(h, parts[0, :N], parts[1, :N],
                 p['W1'], p['b1'].reshape(1, D), p['g1'].reshape(1, D),
                 p['be1'].reshape(1, D),
                 p['W2'], p['b2'].reshape(1, D), p['g2'].reshape(1, D),
                 p['be2'].reshape(1, D))
    return h


# speedup vs baseline: 1.6645x; 1.0695x over previous
import functools

import jax
import jax.numpy as jnp
from jax import lax
from jax.experimental import pallas as pl
from jax.experimental.pallas import tpu as pltpu
from jax.experimental.pallas import tpu_sc as plsc

N = 10000
E = 320000
D = 128
NP = 10240
NW = 32
CHUNK = 125
EPW = E // NW
NCHUNKS = EPW // CHUNK
ROWS_PER_TILE = NP // 16


def _edge_body(h_hbm, src_hbm, dst_hbm, attr_hbm, zeros_hbm, out_hbm,
               acc, src_v, dst_v, rows_v, attr_v, sem, sem_s, sem_i):
    cid = lax.axis_index("c")
    sid = lax.axis_index("s")
    wid = cid * 16 + sid

    pltpu.sync_copy(zeros_hbm.at[pl.ds(sid * ROWS_PER_TILE, ROWS_PER_TILE)],
                    acc.at[pl.ds(sid * ROWS_PER_TILE, ROWS_PER_TILE)])
    plsc.subcore_barrier()

    def chunk_body(i, carry):
        @pl.when(i > 0)
        def _():
            pltpu.make_async_copy(src_hbm.at[wid, 0], src_v, sem_i).wait()

        @pl.when(i == 0)
        def _():
            pltpu.sync_copy(src_hbm.at[wid, 0], src_v)
        pltpu.sync_copy(dst_hbm.at[wid, i], dst_v)

        @pl.when(i > 0)
        def _():
            pltpu.make_async_copy(attr_hbm.at[wid, 0], rows_v, sem_s).wait()

        g = pltpu.async_copy(h_hbm.at[src_v], rows_v, sem)
        pltpu.sync_copy(attr_hbm.at[wid, i], attr_v)
        g.wait()

        @pl.when(i < NCHUNKS - 1)
        def _():
            pltpu.async_copy(src_hbm.at[wid, i + 1], src_v, sem_i)

        def row_body(r, c2):
            for cc in range(D // 16):
                sl = pl.ds(cc * 16, 16)
                v = rows_v[r, sl] + attr_v[r, sl]
                rows_v[r, sl] = jnp.maximum(v, 0.0)
            return c2

        lax.fori_loop(0, CHUNK, row_body, 0)
        pltpu.async_copy(rows_v, acc.at[dst_v], sem_s, add=True)
        return carry

    lax.fori_loop(0, NCHUNKS, chunk_body, 0)
    pltpu.make_async_copy(attr_hbm.at[wid, 0], rows_v, sem_s).wait()
    plsc.subcore_barrier()

    pltpu.sync_copy(acc.at[pl.ds(sid * ROWS_PER_TILE, ROWS_PER_TILE)],
                    out_hbm.at[cid, pl.ds(sid * ROWS_PER_TILE, ROWS_PER_TILE)])


@jax.jit
def _edge_aggregate(h, src, dst, edge_attr, zeros):
    mesh = plsc.VectorSubcoreMesh(core_axis_name="c", subcore_axis_name="s")
    return pl.kernel(
        _edge_body,
        out_type=jax.ShapeDtypeStruct((2, NP, D), jnp.float32),
        mesh=mesh,
        scratch_types=[
            pltpu.VMEM_SHARED((NP, D), jnp.float32),
            pltpu.VMEM((CHUNK,), jnp.int32),
            pltpu.VMEM((CHUNK,), jnp.int32),
            pltpu.VMEM((CHUNK, D), jnp.float32),
            pltpu.VMEM((CHUNK, D), jnp.float32),
            pltpu.SemaphoreType.DMA,
            pltpu.SemaphoreType.DMA,
            pltpu.SemaphoreType.DMA,
        ],
    )(h, src, dst, edge_attr, zeros)


def _mlp_body(h_ref, a0_ref, a1_ref, w1_ref, b1_ref, g1_ref, be1_ref,
              w2_ref, b2_ref, g2_ref, be2_ref, out_ref):
    z = h_ref[...] + a0_ref[...] + a1_ref[...]
    z = jnp.dot(z, w1_ref[...], preferred_element_type=jnp.float32) + b1_ref[...]
    mu = jnp.mean(z, axis=0, keepdims=True)
    var = jnp.mean((z - mu) * (z - mu), axis=0, keepdims=True)
    z = g1_ref[...] * (z - mu) / jnp.sqrt(var + 1e-5) + be1_ref[...]
    z = jnp.maximum(z, 0.0)
    z = jnp.dot(z, w2_ref[...], preferred_element_type=jnp.float32) + b2_ref[...]
    z = jnp.maximum(z, 0.0)
    mu = jnp.mean(z, axis=0, keepdims=True)
    var = jnp.mean((z - mu) * (z - mu), axis=0, keepdims=True)
    z = g2_ref[...] * (z - mu) / jnp.sqrt(var + 1e-5) + be2_ref[...]
    out_ref[...] = jnp.maximum(z, 0.0)


@jax.jit
def _mlp(h, a0, a1, w1, b1, g1, be1, w2, b2, g2, be2):
    return pl.pallas_call(
        _mlp_body,
        out_shape=jax.ShapeDtypeStruct((N, D), jnp.float32),
    )(h, a0, a1, w1, b1, g1, be1, w2, b2, g2, be2)


def kernel(x, edge_index, edge_attr, params):
    src = edge_index[0].astype(jnp.int32).reshape(NW, NCHUNKS, CHUNK)
    dst = edge_index[1].astype(jnp.int32).reshape(NW, NCHUNKS, CHUNK)
    attr = edge_attr.reshape(NW, NCHUNKS, CHUNK, D)
    zeros = jnp.zeros((NP, D), jnp.float32)
    h = x
    for p in params:
        parts = _edge_aggregate(h, src, dst, attr, zeros)
        h = _mlp(h, parts[0, :N], parts[1, :N],
                 p['W1'], p['b1'].reshape(1, D), p['g1'].reshape(1, D),
                 p['be1'].reshape(1, D),
                 p['W2'], p['b2'].reshape(1, D), p['g2'].reshape(1, D),
                 p['be2'].reshape(1, D))
    return h
